# baseline (speedup 1.0000x reference)
import functools
import jax
import jax.numpy as jnp
from jax.experimental import pallas as pl
from jax.experimental.pallas import tpu as pltpu

_CUTOFF = 0.15
_BLK = 256


def _nbr_kernel(a_ref, b_ref, sqc_ref, sqr_ref, out_ref, *, blk):
    a = a_ref[...]
    b = b_ref[...]
    dot = jax.lax.dot_general(
        a, b, (((1,), (0,)), ((), ())),
        preferred_element_type=jnp.float32)
    sqi = sqc_ref[:, 0:1]
    sqj = sqr_ref[0:1, :]
    d2 = (sqi + sqj) + dot
    keep_in = (d2 < _CUTOFF * _CUTOFF) & (d2 > 0.0)
    out_ref[...] = jnp.where(keep_in, d2 * jax.lax.rsqrt(d2), 0.0)
    j0 = pl.program_id(0) * blk
    r = jax.lax.broadcasted_iota(jnp.int32, (blk, blk), 0)
    c = jax.lax.broadcasted_iota(jnp.int32, (blk, blk), 1)
    keep = jnp.where(r == c, 0.0, 1.0)
    out_ref[:, pl.ds(j0, blk)] = out_ref[:, pl.ds(j0, blk)] * keep


def kernel(positions):
    n = positions.shape[0]
    sq = jnp.sum(positions * positions, axis=1, keepdims=True)
    pb = positions.astype(jnp.bfloat16)
    zb = jnp.zeros((n, 13), jnp.bfloat16)
    a = jnp.concatenate([pb, zb], axis=1)
    b = jnp.concatenate([-2.0 * pb, zb], axis=1).T
    sqc = jnp.pad(sq, ((0, 0), (0, 7)))
    sqr = sqc.T
    return pl.pallas_call(
        functools.partial(_nbr_kernel, blk=_BLK),
        grid=(n // _BLK,),
        in_specs=[
            pl.BlockSpec((_BLK, 16), lambda i: (i, 0)),
            pl.BlockSpec((16, n), lambda i: (0, 0)),
            pl.BlockSpec((_BLK, 8), lambda i: (i, 0)),
            pl.BlockSpec((8, n), lambda i: (0, 0)),
        ],
        out_specs=pl.BlockSpec((_BLK, n), lambda i: (i, 0)),
        out_shape=jax.ShapeDtypeStruct((n, n), jnp.float32),
        compiler_params=pltpu.CompilerParams(
            dimension_semantics=("parallel",)),
    )(a, b, sqc, sqr)


# speedup vs baseline: 1.1908x; 1.1908x over previous
import functools
import jax
import jax.numpy as jnp
from jax.experimental import pallas as pl
from jax.experimental.pallas import tpu as pltpu

_CUTOFF = 0.15
_BLK = 512


def _nbr_kernel(p_ref, pt_ref, out_ref, *, blk):
    a8 = p_ref[...]
    bt8 = pt_ref[...]
    sqi = a8[:, 3:4]
    sqj = bt8[3:4, :]
    a = a8.astype(jnp.bfloat16)
    s = jnp.where(
        jax.lax.broadcasted_iota(jnp.int32, (8, 1), 0) < 3, -2.0, 0.0)
    b = (bt8 * s).astype(jnp.bfloat16)
    dot = jax.lax.dot_general(
        a, b, (((1,), (0,)), ((), ())),
        preferred_element_type=jnp.float32)
    d2 = (sqi + sqj) + dot
    keep_in = (d2 < _CUTOFF * _CUTOFF) & (d2 > 0.0)
    out_ref[...] = jnp.where(keep_in, d2 * jax.lax.rsqrt(d2), 0.0)
    j0 = pl.program_id(0) * blk
    r = jax.lax.broadcasted_iota(jnp.int32, (blk, blk), 0)
    c = jax.lax.broadcasted_iota(jnp.int32, (blk, blk), 1)
    keep = jnp.where(r == c, 0.0, 1.0)
    out_ref[:, pl.ds(j0, blk)] = out_ref[:, pl.ds(j0, blk)] * keep


def kernel(positions):
    n = positions.shape[0]
    sq = jnp.sum(positions * positions, axis=1, keepdims=True)
    p = jnp.concatenate([positions, sq, jnp.zeros((n, 4), jnp.float32)], 1)
    pt = p.T
    return pl.pallas_call(
        functools.partial(_nbr_kernel, blk=_BLK),
        grid=(n // _BLK,),
        in_specs=[
            pl.BlockSpec((_BLK, 8), lambda i: (i, 0)),
            pl.BlockSpec((8, n), lambda i: (0, 0)),
        ],
        out_specs=pl.BlockSpec((_BLK, n), lambda i: (i, 0)),
        out_shape=jax.ShapeDtypeStruct((n, n), jnp.float32),
        compiler_params=pltpu.CompilerParams(
            dimension_semantics=("parallel",)),
    )(p, pt)
